# baseline (speedup 1.0000x reference)
import functools

import jax
import jax.numpy as jnp
from jax import lax
from jax.experimental import pallas as pl
from jax.experimental.pallas import tpu as pltpu
from jax.experimental.pallas import tpu_sc as plsc

V = 100000
P = 128
NB = 4096
S = 50
NW = 32
PPT = P // NW
L = 16
UNROLL = 16
NV = NB // (L * UNROLL)


def _make_gather():
    mesh = plsc.VectorSubcoreMesh(core_axis_name="c", subcore_axis_name="s")

    @functools.partial(
        pl.kernel,
        mesh=mesh,
        compiler_params=pltpu.CompilerParams(needs_layout_passes=False),
        out_type=jax.ShapeDtypeStruct((S * P, NB), jnp.float32),
        scratch_types=[
            pltpu.VMEM((V,), jnp.float32),
            pltpu.VMEM((NB,), jnp.int32),
            pltpu.VMEM((NB,), jnp.int32),
            pltpu.VMEM((NB,), jnp.float32),
            pltpu.VMEM((NB,), jnp.float32),
            pltpu.SemaphoreType.DMA,
            pltpu.SemaphoreType.DMA,
            pltpu.SemaphoreType.DMA,
            pltpu.SemaphoreType.DMA,
        ],
    )
    def gather_kernel(xT, idxT, outT, plane, ia, ib, oa, ob,
                      sia, sib, soa, sob):
        wid = lax.axis_index("s") * 2 + lax.axis_index("c")

        def gather_row(idxb, outb):
            @plsc.parallel_loop(0, NB, L, unroll=UNROLL)
            def body(i):
                ids = idxb[pl.ds(i, L)]
                outb[pl.ds(i, L)] = plsc.load_gather(plane, [ids])

        def wait_idx(sem):
            pltpu.make_async_copy(idxT.at[0], ia, sem).wait()

        def wait_out(sem):
            pltpu.make_async_copy(oa, outT.at[0], sem).wait()

        for pi in range(PPT):
            p = wid * PPT + pi
            pltpu.sync_copy(xT.at[p], plane)
            pltpu.async_copy(idxT.at[0], ia, sia)
            pltpu.async_copy(idxT.at[1], ib, sib)

            wait_idx(sia)
            gather_row(ia, oa)
            pltpu.async_copy(oa, outT.at[p], soa)
            pltpu.async_copy(idxT.at[2], ia, sia)
            wait_idx(sib)
            gather_row(ib, ob)
            pltpu.async_copy(ob, outT.at[P + p], sob)
            pltpu.async_copy(idxT.at[3], ib, sib)

            def pair(g, carry):
                s0 = 2 * g
                wait_idx(sia)
                wait_out(soa)
                gather_row(ia, oa)
                pltpu.async_copy(oa, outT.at[s0 * P + p], soa)
                pltpu.async_copy(idxT.at[s0 + 2], ia, sia)
                wait_idx(sib)
                wait_out(sob)
                gather_row(ib, ob)
                pltpu.async_copy(ob, outT.at[(s0 + 1) * P + p], sob)
                pltpu.async_copy(idxT.at[s0 + 3], ib, sib)
                return carry

            lax.fori_loop(1, S // 2 - 1, pair, 0)

            wait_idx(sia)
            wait_out(soa)
            gather_row(ia, oa)
            pltpu.async_copy(oa, outT.at[(S - 2) * P + p], soa)
            wait_idx(sib)
            wait_out(sob)
            gather_row(ib, ob)
            pltpu.async_copy(ob, outT.at[(S - 1) * P + p], sob)
            wait_out(soa)
            wait_out(sob)

    return gather_kernel


_gather = _make_gather()


@jax.jit
def kernel(x, index):
    b, s = index.shape
    xT = x.transpose(1, 2, 0).reshape(P, V)
    idxT = index.astype(jnp.int32).T
    outT = _gather(xT, idxT)
    return outT.reshape(s, 16, 8, b).transpose(3, 0, 1, 2)


# speedup vs baseline: 1.6209x; 1.6209x over previous
import functools

import jax
import jax.numpy as jnp
from jax import lax
from jax.experimental import pallas as pl
from jax.experimental.pallas import tpu as pltpu
from jax.experimental.pallas import tpu_sc as plsc

V = 100000
P = 128
NB = 4096
S = 50
NW = 32
PPT = P // NW
L = 16
UNROLL = 8
NV = NB // (L * UNROLL)


def _make_gather():
    mesh = plsc.VectorSubcoreMesh(core_axis_name="c", subcore_axis_name="s")

    @functools.partial(
        pl.kernel,
        mesh=mesh,
        compiler_params=pltpu.CompilerParams(needs_layout_passes=False),
        out_type=jax.ShapeDtypeStruct((S * P, NB), jnp.float32),
        scratch_types=[
            pltpu.VMEM((V,), jnp.float32),
            pltpu.VMEM((NB,), jnp.int32),
            pltpu.VMEM((NB,), jnp.int32),
            pltpu.VMEM((NB,), jnp.float32),
            pltpu.VMEM((NB,), jnp.float32),
            pltpu.VMEM_SHARED((S, NB), jnp.int32),
            pltpu.SemaphoreType.DMA,
            pltpu.SemaphoreType.DMA,
            pltpu.SemaphoreType.DMA,
            pltpu.SemaphoreType.DMA,
        ],
    )
    def gather_kernel(xT, idxT, outT, plane, ia, ib, oa, ob, idx_sh,
                      sia, sib, soa, sob):
        wid = lax.axis_index("s") * 2 + lax.axis_index("c")
        sid = lax.axis_index("s")

        for k in range(4):
            r = sid * 4 + k
            @pl.when(r < S)
            def _():
                pltpu.sync_copy(idxT.at[r], idx_sh.at[r])
        plsc.subcore_barrier()

        def gather_row(idxb, outb):
            @plsc.parallel_loop(0, NB, L, unroll=UNROLL)
            def body(i):
                ids = idxb[pl.ds(i, L)]
                outb[pl.ds(i, L)] = plsc.load_gather(plane, [ids])

        def wait_idx(sem):
            pltpu.make_async_copy(idx_sh.at[0], ia, sem).wait()

        def wait_out(sem):
            pltpu.make_async_copy(oa, outT.at[0], sem).wait()

        for pi in range(PPT):
            p = wid * PPT + pi
            pltpu.sync_copy(xT.at[p], plane)
            pltpu.async_copy(idx_sh.at[0], ia, sia)
            pltpu.async_copy(idx_sh.at[1], ib, sib)

            wait_idx(sia)
            gather_row(ia, oa)
            pltpu.async_copy(oa, outT.at[p], soa)
            pltpu.async_copy(idx_sh.at[2], ia, sia)
            wait_idx(sib)
            gather_row(ib, ob)
            pltpu.async_copy(ob, outT.at[P + p], sob)
            pltpu.async_copy(idx_sh.at[3], ib, sib)

            def pair(g, carry):
                s0 = 2 * g
                wait_idx(sia)
                wait_out(soa)
                gather_row(ia, oa)
                pltpu.async_copy(oa, outT.at[s0 * P + p], soa)
                pltpu.async_copy(idx_sh.at[s0 + 2], ia, sia)
                wait_idx(sib)
                wait_out(sob)
                gather_row(ib, ob)
                pltpu.async_copy(ob, outT.at[(s0 + 1) * P + p], sob)
                pltpu.async_copy(idx_sh.at[s0 + 3], ib, sib)
                return carry

            lax.fori_loop(1, S // 2 - 1, pair, 0)

            wait_idx(sia)
            wait_out(soa)
            gather_row(ia, oa)
            pltpu.async_copy(oa, outT.at[(S - 2) * P + p], soa)
            wait_idx(sib)
            wait_out(sob)
            gather_row(ib, ob)
            pltpu.async_copy(ob, outT.at[(S - 1) * P + p], sob)
            wait_out(soa)
            wait_out(sob)

    return gather_kernel


_gather = _make_gather()


@jax.jit
def kernel(x, index):
    b, s = index.shape
    xT = x.transpose(1, 2, 0).reshape(P, V)
    idxT = index.astype(jnp.int32).T
    outT = _gather(xT, idxT)
    return outT.reshape(s, 16, 8, b).transpose(3, 0, 1, 2)
